# baseline (speedup 1.0000x reference)
import jax
import jax.numpy as jnp
from jax.experimental import pallas as pl
from jax.experimental.pallas import tpu as pltpu
from jax.experimental.pallas import tpu_sc as plsc

N = 10000
E = 320000
HID = 128
ALPHA = 0.2

NC = 2
NS = 16
NW = NC * NS
L = 16

C = 64
E2 = E + N
NCH = 328
EP = NS * C * NCH
NCHR = 316
EPR = NS * C * NCHR
NCHD = 160
EPD = NW * C * NCHD
NP2 = 10112
OWN = NP2 // NS

ET = E // NW


def _dgi_perm():
    return jax.random.permutation(jax.random.key(1), N).astype(jnp.int32)


_MESH = plsc.VectorSubcoreMesh(
    core_axis_name="c", subcore_axis_name="s", num_cores=NC, num_subcores=NS
)
_SC_PARAMS = pltpu.CompilerParams(needs_layout_passes=False)


def _deg_body(dst_hbm, ew_hbm, out_hbm, dstv, ewv, deg):
    c = jax.lax.axis_index("c")
    s = jax.lax.axis_index("s")
    wid = s * NC + c

    def zero(i, _):
        deg[pl.ds(i * L, L)] = jnp.zeros((L,), jnp.float32)
        return 0

    jax.lax.fori_loop(0, N // L, zero, 0)
    pltpu.sync_copy(dst_hbm.at[pl.ds(wid * ET, ET)], dstv)
    pltpu.sync_copy(ew_hbm.at[pl.ds(wid * ET, ET)], ewv)

    def step(e, _):
        dv = dstv[pl.ds(e * L, L)]
        wv = ewv[pl.ds(e * L, L)]
        plsc.addupdate_scatter(deg, [dv], wv)
        return 0

    jax.lax.fori_loop(0, ET // L, step, 0)
    pltpu.sync_copy(deg, out_hbm.at[wid])


def _k_deg(dst, ew):
    f = pl.kernel(
        _deg_body,
        out_type=jax.ShapeDtypeStruct((NW, N), jnp.float32),
        mesh=_MESH,
        compiler_params=_SC_PARAMS,
        scratch_types=[
            pltpu.VMEM((ET,), jnp.int32),
            pltpu.VMEM((ET,), jnp.float32),
            pltpu.VMEM((N,), jnp.float32),
        ],
    )
    return f(dst, ew)


def _norm_body(src_hbm, dst_hbm, ew_hbm, dinv_hbm, perm_hbm,
               norm_hbm, psrc_hbm,
               srcv, dstv, ewv, dinvv, permv, normv, psrcv):
    c = jax.lax.axis_index("c")
    s = jax.lax.axis_index("s")
    wid = s * NC + c
    pltpu.sync_copy(src_hbm.at[pl.ds(wid * ET, ET)], srcv)
    pltpu.sync_copy(dst_hbm.at[pl.ds(wid * ET, ET)], dstv)
    pltpu.sync_copy(ew_hbm.at[pl.ds(wid * ET, ET)], ewv)
    pltpu.sync_copy(dinv_hbm, dinvv)
    pltpu.sync_copy(perm_hbm, permv)

    def step(e, _):
        sv = srcv[pl.ds(e * L, L)]
        dv = dstv[pl.ds(e * L, L)]
        a = plsc.load_gather(dinvv, [sv])
        b = plsc.load_gather(dinvv, [dv])
        normv[pl.ds(e * L, L)] = a * ewv[pl.ds(e * L, L)] * b
        psrcv[pl.ds(e * L, L)] = plsc.load_gather(permv, [sv])
        return 0

    jax.lax.fori_loop(0, ET // L, step, 0)
    pltpu.sync_copy(normv, norm_hbm.at[pl.ds(wid * ET, ET)])
    pltpu.sync_copy(psrcv, psrc_hbm.at[pl.ds(wid * ET, ET)])


def _k_norm(src, dst, ew, dinv, perm):
    f = pl.kernel(
        _norm_body,
        out_type=[
            jax.ShapeDtypeStruct((E,), jnp.float32),
            jax.ShapeDtypeStruct((E,), jnp.int32),
        ],
        mesh=_MESH,
        compiler_params=_SC_PARAMS,
        scratch_types=[
            pltpu.VMEM((ET,), jnp.int32),
            pltpu.VMEM((ET,), jnp.int32),
            pltpu.VMEM((ET,), jnp.float32),
            pltpu.VMEM((N,), jnp.float32),
            pltpu.VMEM((N,), jnp.int32),
            pltpu.VMEM((ET,), jnp.float32),
            pltpu.VMEM((ET,), jnp.int32),
        ],
    )
    return f(src, dst, ew, dinv, perm)


NBUF = 4


def _make_prop_body(nch):
    assert nch % NBUF == 0

    def _prop_body(table_hbm, meta_hbm, out_hbm,
                   acc, metag, rows0, rows1, rows2, rows3,
                   semd0, semd1, semd2, semd3,
                   semg0, semg1, semg2, semg3,
                   sems0, sems1, sems2, sems3):
        rows = (rows0, rows1, rows2, rows3)
        semd = (semd0, semd1, semd2, semd3)
        semg = (semg0, semg1, semg2, semg3)
        sems = (sems0, sems1, sems2, sems3)
        c = jax.lax.axis_index("c")
        s = jax.lax.axis_index("s")
        trow = (c * NS + s) * nch

        def zrow(r, _):
            for j in range(HID // L):
                rows0[r, pl.ds(j * L, L)] = jnp.zeros((L,), jnp.float32)
            return 0

        jax.lax.fori_loop(0, C, zrow, 0)
        base = s * OWN
        for k in range(OWN // C):
            pltpu.sync_copy(rows0.at[pl.ds(0, C)],
                            acc.at[pl.ds(base + k * C, C)])
        rem = OWN % C
        if rem:
            pltpu.sync_copy(rows0.at[pl.ds(0, rem)],
                            acc.at[pl.ds(base + (OWN // C) * C, rem)])
        plsc.subcore_barrier()

        def issue_meta(b, ch):
            pltpu.async_copy(meta_hbm.at[trow + ch], metag.at[b], semd[b])

        def wait_meta(b):
            pltpu.make_async_copy(meta_hbm.at[trow], metag.at[b],
                                  semd[b]).wait()

        def issue_gather(b):
            pltpu.async_copy(table_hbm.at[metag.at[b, 0]], rows[b], semg[b])

        def wait_gather(b):
            pltpu.make_async_copy(table_hbm.at[metag.at[b, 0]],
                                  rows[b], semg[b]).wait()

        def issue_scatter(b):
            pltpu.async_copy(rows[b], acc.at[metag.at[b, 1]], sems[b],
                             add=True)

        def wait_scatter(b):
            pltpu.make_async_copy(rows[b], acc.at[metag.at[b, 1]],
                                  sems[b]).wait()

        issue_meta(0, 0)
        issue_meta(1, 1)
        wait_meta(0)
        issue_gather(0)

        def quad(i, _):
            for b in range(NBUF):
                ch = i * NBUF + b
                b1 = (b + 1) % NBUF
                b2 = (b + 2) % NBUF

                @pl.when(ch >= 2)
                def _():
                    wait_scatter(b2)

                @pl.when(ch + 2 < nch)
                def _():
                    issue_meta(b2, ch + 2)

                @pl.when(ch + 1 < nch)
                def _():
                    wait_meta(b1)
                    issue_gather(b1)

                wait_gather(b)

                @plsc.parallel_loop(0, C, 1, unroll=4)
                def scale(e):
                    nbits = plsc.load_gather(metag.at[b, 2],
                                             [jnp.full((L,), e, jnp.int32)])
                    nb = plsc.bitcast(nbits, jnp.float32)
                    for j in range(HID // L):
                        sl = pl.ds(j * L, L)
                        rows[b][e, sl] = rows[b][e, sl] * nb

                issue_scatter(b)
            return 0

        jax.lax.fori_loop(0, nch // NBUF, quad, 0)
        wait_scatter((nch - 2) % NBUF)
        wait_scatter((nch - 1) % NBUF)
        plsc.subcore_barrier()
        pltpu.sync_copy(acc.at[pl.ds(base, OWN)],
                        out_hbm.at[c, pl.ds(base, OWN)])

    return _prop_body


def _k_prop(table, meta, nch):
    f = pl.kernel(
        _make_prop_body(nch),
        out_type=jax.ShapeDtypeStruct((NC, NP2, HID), jnp.float32),
        mesh=_MESH,
        compiler_params=_SC_PARAMS,
        scratch_types=[
            pltpu.VMEM_SHARED((NP2, HID), jnp.float32),
            pltpu.VMEM((NBUF, 3, C), jnp.int32),
            pltpu.VMEM((C, HID), jnp.float32),
            pltpu.VMEM((C, HID), jnp.float32),
            pltpu.VMEM((C, HID), jnp.float32),
            pltpu.VMEM((C, HID), jnp.float32),
            pltpu.SemaphoreType.DMA,
            pltpu.SemaphoreType.DMA,
            pltpu.SemaphoreType.DMA,
            pltpu.SemaphoreType.DMA,
            pltpu.SemaphoreType.DMA,
            pltpu.SemaphoreType.DMA,
            pltpu.SemaphoreType.DMA,
            pltpu.SemaphoreType.DMA,
            pltpu.SemaphoreType.DMA,
            pltpu.SemaphoreType.DMA,
            pltpu.SemaphoreType.DMA,
            pltpu.SemaphoreType.DMA,
        ],
    )
    return f(table, meta)


def _dinv_body(part_ref, dinv_ref, selfnorm_ref):
    deg = jnp.sum(part_ref[...], axis=0, keepdims=True) + 1.0
    dinv = jax.lax.rsqrt(deg)
    dinv_ref[...] = dinv
    selfnorm_ref[...] = dinv * dinv


def _k_dinv(partials):
    return pl.pallas_call(
        _dinv_body,
        out_shape=[
            jax.ShapeDtypeStruct((1, N), jnp.float32),
            jax.ShapeDtypeStruct((1, N), jnp.float32),
        ],
    )(partials)


def _mm_body(x_ref, w_ref, o_ref):
    o_ref[...] = jax.lax.dot_general(
        x_ref[...], w_ref[...], (((1,), (0,)), ((), ())),
        preferred_element_type=jnp.float32)


def _k_mm(x, w):
    return pl.pallas_call(
        _mm_body,
        out_shape=jax.ShapeDtypeStruct((x.shape[0], w.shape[1]), jnp.float32),
    )(x, w)


def _bias_mm_body(o_ref, b_ref, w_ref, h_ref):
    h = o_ref[:, :N, :].reshape(NC * N, HID) + b_ref[...]
    h_ref[...] = jax.lax.dot_general(
        h, w_ref[...], (((1,), (0,)), ((), ())),
        preferred_element_type=jnp.float32)


def _k_bias_mm(o, b, w):
    return pl.pallas_call(
        _bias_mm_body,
        out_shape=jax.ShapeDtypeStruct((NC * N, HID), jnp.float32),
    )(o, b.reshape(1, HID), w)


def _bias_mm_sn_body(o_ref, t_ref, sn_ref, b_ref, w_ref, h_ref):
    h = (o_ref[:, :N, :].reshape(NC * N, HID)
         + jnp.tile(sn_ref[...], (NC, 1)) * t_ref[...]
         + b_ref[...])
    h_ref[...] = jax.lax.dot_general(
        h, w_ref[...], (((1,), (0,)), ((), ())),
        preferred_element_type=jnp.float32)


def _k_bias_mm_sn(o, t, sn_col, b, w):
    return pl.pallas_call(
        _bias_mm_sn_body,
        out_shape=jax.ShapeDtypeStruct((NC * N, HID), jnp.float32),
    )(o, t, sn_col, b.reshape(1, HID), w)


def _post_body(o_ref, t_ref, sn_ref, b_ref, a_ref, wc_ref,
               z_ref, hdec_ref, sum_ref):
    h = (o_ref[:, :N, :]
         + (jnp.tile(sn_ref[...], (NC, 1)) * t_ref[...]).reshape(NC, N, HID)
         + b_ref[...][None])
    z = jnp.where(h >= 0, h, a_ref[...][None] * h)
    z_ref[...] = z
    zp = z[0]
    hdec_ref[...] = jax.lax.dot_general(
        zp, wc_ref[...], (((1,), (0,)), ((), ())),
        preferred_element_type=jnp.float32)
    m = jnp.mean(zp, axis=0, keepdims=True)
    sum_ref[...] = 1.0 / (1.0 + jnp.exp(-m))


def _k_post(o, t, sn_col, b, a, wc):
    return pl.pallas_call(
        _post_body,
        out_shape=[
            jax.ShapeDtypeStruct((NC, N, HID), jnp.float32),
            jax.ShapeDtypeStruct((N, HID), jnp.float32),
            jax.ShapeDtypeStruct((1, HID), jnp.float32),
        ],
    )(o, t, sn_col, b.reshape(1, HID), a.reshape(1, HID), wc)


def _fin_body(p_ref, t_ref, sn_ref, bc_ref, mu_ref, xr_ref, q_ref):
    xr = (p_ref[0, :N, :] + p_ref[1, :N, :]
          + sn_ref[...] * t_ref[...] + bc_ref[...])
    xr_ref[...] = xr
    mu = mu_ref[...]
    d2 = (jnp.sum(xr * xr, axis=1, keepdims=True)
          + jnp.sum(mu * mu, axis=1)[None, :]
          - 2.0 * jax.lax.dot_general(xr, mu, (((1,), (1,)), ((), ())),
                                      preferred_element_type=jnp.float32))
    q = 1.0 / (1.0 + d2 / ALPHA + 1e-08)
    q = q ** (ALPHA + 1.0) / 2.0
    q_ref[...] = q / jnp.sum(q, axis=1, keepdims=True)


def _k_fin(p, t, sn_col, bc, mu):
    K = mu.shape[0]
    return pl.pallas_call(
        _fin_body,
        out_shape=[
            jax.ShapeDtypeStruct((N, HID), jnp.float32),
            jax.ShapeDtypeStruct((N, K), jnp.float32),
        ],
    )(p, t, sn_col, bc.reshape(1, HID), mu)


def kernel(x, edge_index, edge_attr, W1, b1, W2, b2, W3, b3, W4, b4, prelu_a, Wc, bc, mu):
    src = edge_index[0].astype(jnp.int32)
    dst = edge_index[1].astype(jnp.int32)
    perm = _dgi_perm()

    partials = _k_deg(dst, edge_attr)
    dinv2d, selfnorm2d = _k_dinv(partials)
    dinv = dinv2d[0]
    selfnorm = selfnorm2d[0]
    norm_e, psrc = _k_norm(src, dst, edge_attr, dinv, perm)

    sn_col = selfnorm.reshape(N, 1)

    sl = jnp.arange(N, dtype=jnp.int32)
    pz1 = jnp.zeros((EP - E2,), jnp.int32)
    pf1 = jnp.zeros((EP - E2,), jnp.float32)
    pzr = jnp.zeros((EPR - E,), jnp.int32)
    pfr = jnp.zeros((EPR - E,), jnp.float32)
    pzd = jnp.zeros((EPD - E,), jnp.int32)
    pfd = jnp.zeros((EPD - E,), jnp.float32)

    def mk_meta(s_, d_, n_):
        return jnp.concatenate(
            [s_.reshape(-1, 1, C), d_.reshape(-1, 1, C),
             jax.lax.bitcast_convert_type(n_, jnp.int32).reshape(-1, 1, C)],
            axis=1)

    meta1 = mk_meta(
        jnp.concatenate([src, sl, pz1, psrc, perm, pz1]),
        jnp.concatenate([dst, sl, pz1, dst, sl, pz1]),
        jnp.concatenate([norm_e, selfnorm, pf1, norm_e, selfnorm, pf1]))
    metaR = mk_meta(
        jnp.concatenate([src, pzr, src + N, pzr]),
        jnp.concatenate([dst, pzr, dst, pzr]),
        jnp.concatenate([norm_e, pfr, norm_e, pfr]))
    metaD = mk_meta(
        jnp.concatenate([src, pzd]),
        jnp.concatenate([dst, pzd]),
        jnp.concatenate([norm_e, pfd]))

    xw = _k_mm(x, W1)
    o1 = _k_prop(xw, meta1, NCH)
    h2 = _k_bias_mm(o1, b1, W2)
    o2 = _k_prop(h2, metaR, NCHR)
    h3 = _k_bias_mm_sn(o2, h2, sn_col, b2, W3)
    o3 = _k_prop(h3, metaR, NCHR)
    h4 = _k_bias_mm_sn(o3, h3, sn_col, b3, W4)
    o4 = _k_prop(h4, metaR, NCHR)
    z2, hdec, sum2d = _k_post(o4, h4, sn_col, b4, prelu_a, Wc)
    od = _k_prop(hdec, metaD, NCHD)
    xr, q = _k_fin(od, hdec, sn_col, bc, mu)

    pos_z = z2[0]
    neg_z = z2[1]
    summary = sum2d[0]
    return (pos_z, neg_z, summary, xr, q)


# speedup vs baseline: 1.4904x; 1.4904x over previous
import jax
import jax.numpy as jnp
from jax.experimental import pallas as pl
from jax.experimental.pallas import tpu as pltpu
from jax.experimental.pallas import tpu_sc as plsc

N = 10000
E = 320000
HID = 128
ALPHA = 0.2

NC = 2
NS = 16
NW = NC * NS
L = 16

C = 88
E2 = E + N
NCH = 236
EP = NS * C * NCH
NCHR = 228
EPR = NS * C * NCHR
NCHD = 116
EPD = NW * C * NCHD
NP2 = 10112
OWN = NP2 // NS

ET = E // NW


def _dgi_perm():
    return jax.random.permutation(jax.random.key(1), N).astype(jnp.int32)


_MESH = plsc.VectorSubcoreMesh(
    core_axis_name="c", subcore_axis_name="s", num_cores=NC, num_subcores=NS
)
_SC_PARAMS = pltpu.CompilerParams(needs_layout_passes=False)


def _deg_body(dst_hbm, ew_hbm, out_hbm, dstv, ewv, deg):
    c = jax.lax.axis_index("c")
    s = jax.lax.axis_index("s")
    wid = s * NC + c

    def zero(i, _):
        deg[pl.ds(i * L, L)] = jnp.zeros((L,), jnp.float32)
        return 0

    jax.lax.fori_loop(0, N // L, zero, 0)
    pltpu.sync_copy(dst_hbm.at[pl.ds(wid * ET, ET)], dstv)
    pltpu.sync_copy(ew_hbm.at[pl.ds(wid * ET, ET)], ewv)

    def step(e, _):
        dv = dstv[pl.ds(e * L, L)]
        wv = ewv[pl.ds(e * L, L)]
        plsc.addupdate_scatter(deg, [dv], wv)
        return 0

    jax.lax.fori_loop(0, ET // L, step, 0)
    pltpu.sync_copy(deg, out_hbm.at[wid])


def _k_deg(dst, ew):
    f = pl.kernel(
        _deg_body,
        out_type=jax.ShapeDtypeStruct((NW, N), jnp.float32),
        mesh=_MESH,
        compiler_params=_SC_PARAMS,
        scratch_types=[
            pltpu.VMEM((ET,), jnp.int32),
            pltpu.VMEM((ET,), jnp.float32),
            pltpu.VMEM((N,), jnp.float32),
        ],
    )
    return f(dst, ew)


def _norm_body(src_hbm, dst_hbm, ew_hbm, dinv_hbm, perm_hbm,
               norm_hbm, psrc_hbm,
               srcv, dstv, ewv, dinvv, permv, normv, psrcv):
    c = jax.lax.axis_index("c")
    s = jax.lax.axis_index("s")
    wid = s * NC + c
    pltpu.sync_copy(src_hbm.at[pl.ds(wid * ET, ET)], srcv)
    pltpu.sync_copy(dst_hbm.at[pl.ds(wid * ET, ET)], dstv)
    pltpu.sync_copy(ew_hbm.at[pl.ds(wid * ET, ET)], ewv)
    pltpu.sync_copy(dinv_hbm, dinvv)
    pltpu.sync_copy(perm_hbm, permv)

    def step(e, _):
        sv = srcv[pl.ds(e * L, L)]
        dv = dstv[pl.ds(e * L, L)]
        a = plsc.load_gather(dinvv, [sv])
        b = plsc.load_gather(dinvv, [dv])
        normv[pl.ds(e * L, L)] = a * ewv[pl.ds(e * L, L)] * b
        psrcv[pl.ds(e * L, L)] = plsc.load_gather(permv, [sv])
        return 0

    jax.lax.fori_loop(0, ET // L, step, 0)
    pltpu.sync_copy(normv, norm_hbm.at[pl.ds(wid * ET, ET)])
    pltpu.sync_copy(psrcv, psrc_hbm.at[pl.ds(wid * ET, ET)])


def _k_norm(src, dst, ew, dinv, perm):
    f = pl.kernel(
        _norm_body,
        out_type=[
            jax.ShapeDtypeStruct((E,), jnp.float32),
            jax.ShapeDtypeStruct((E,), jnp.int32),
        ],
        mesh=_MESH,
        compiler_params=_SC_PARAMS,
        scratch_types=[
            pltpu.VMEM((ET,), jnp.int32),
            pltpu.VMEM((ET,), jnp.int32),
            pltpu.VMEM((ET,), jnp.float32),
            pltpu.VMEM((N,), jnp.float32),
            pltpu.VMEM((N,), jnp.int32),
            pltpu.VMEM((ET,), jnp.float32),
            pltpu.VMEM((ET,), jnp.int32),
        ],
    )
    return f(src, dst, ew, dinv, perm)


NBUF = 4


def _make_prop_body(nch):
    assert nch % NBUF == 0

    def _prop_body(table_hbm, src_hbm, dst_hbm, nrm_hbm, out_hbm,
                   acc, srcg, dstg, nrmg, rows0, rows1, rows2, rows3,
                   semd0, semd1, semd2, semd3,
                   semg0, semg1, semg2, semg3,
                   sems0, sems1, sems2, sems3,
                   semh0, semh1, semh2, semh3):
        rows = (rows0, rows1, rows2, rows3)
        semd = (semd0, semd1, semd2, semd3)
        semg = (semg0, semg1, semg2, semg3)
        sems = (sems0, sems1, sems2, sems3)
        sems2g = (semh0, semh1, semh2, semh3)
        c = jax.lax.axis_index("c")
        s = jax.lax.axis_index("s")
        toff = (c * NS + s) * (nch * C)

        def zrow(r, _):
            for j in range(HID // L):
                rows0[r, pl.ds(j * L, L)] = jnp.zeros((L,), jnp.float32)
            return 0

        jax.lax.fori_loop(0, C, zrow, 0)
        base = s * OWN
        for k in range(OWN // C):
            pltpu.sync_copy(rows0.at[pl.ds(0, C)],
                            acc.at[pl.ds(base + k * C, C)])
        rem = OWN % C
        if rem:
            pltpu.sync_copy(rows0.at[pl.ds(0, rem)],
                            acc.at[pl.ds(base + (OWN // C) * C, rem)])
        plsc.subcore_barrier()

        def issue_meta(b, ch):
            off = toff + ch * C
            pltpu.async_copy(src_hbm.at[pl.ds(off, C)], srcg.at[b], semd[b])
            pltpu.async_copy(dst_hbm.at[pl.ds(off, C)], dstg.at[b], semd[b])
            pltpu.async_copy(nrm_hbm.at[pl.ds(off, C)], nrmg.at[b], semd[b])

        def wait_meta(b):
            pltpu.make_async_copy(src_hbm.at[pl.ds(toff, C)],
                                  srcg.at[b], semd[b]).wait()
            pltpu.make_async_copy(dst_hbm.at[pl.ds(toff, C)],
                                  dstg.at[b], semd[b]).wait()
            pltpu.make_async_copy(nrm_hbm.at[pl.ds(toff, C)],
                                  nrmg.at[b], semd[b]).wait()

        H2 = C // 2

        def issue_gather(b):
            pltpu.async_copy(table_hbm.at[srcg.at[b, pl.ds(0, H2)]],
                             rows[b].at[pl.ds(0, H2)], semg[b])
            pltpu.async_copy(table_hbm.at[srcg.at[b, pl.ds(H2, H2)]],
                             rows[b].at[pl.ds(H2, H2)], sems2g[b])

        def wait_gather(b):
            pltpu.make_async_copy(table_hbm.at[srcg.at[b, pl.ds(0, H2)]],
                                  rows[b].at[pl.ds(0, H2)], semg[b]).wait()
            pltpu.make_async_copy(table_hbm.at[srcg.at[b, pl.ds(0, H2)]],
                                  rows[b].at[pl.ds(H2, H2)], sems2g[b]).wait()

        def issue_scatter(b):
            pltpu.async_copy(rows[b], acc.at[dstg.at[b]], sems[b], add=True)

        def wait_scatter(b):
            pltpu.make_async_copy(rows[b], acc.at[dstg.at[b]], sems[b]).wait()

        issue_meta(0, 0)
        issue_meta(1, 1)
        wait_meta(0)
        issue_gather(0)

        def quad(i, _):
            for b in range(NBUF):
                ch = i * NBUF + b
                b1 = (b + 1) % NBUF
                b2 = (b + 2) % NBUF

                @pl.when(ch >= 2)
                def _():
                    wait_scatter(b2)

                @pl.when(ch + 2 < nch)
                def _():
                    issue_meta(b2, ch + 2)

                @pl.when(ch + 1 < nch)
                def _():
                    wait_meta(b1)
                    issue_gather(b1)

                wait_gather(b)

                @plsc.parallel_loop(0, C, 1, unroll=4)
                def scale(e):
                    nb = plsc.load_gather(nrmg.at[b],
                                          [jnp.full((L,), e, jnp.int32)])
                    for j in range(HID // L):
                        sl = pl.ds(j * L, L)
                        rows[b][e, sl] = rows[b][e, sl] * nb

                issue_scatter(b)
            return 0

        jax.lax.fori_loop(0, nch // NBUF, quad, 0)
        wait_scatter((nch - 2) % NBUF)
        wait_scatter((nch - 1) % NBUF)
        plsc.subcore_barrier()
        pltpu.sync_copy(acc.at[pl.ds(base, OWN)],
                        out_hbm.at[c, pl.ds(base, OWN)])

    return _prop_body


def _k_prop(table, srcF, dstF, nrmF, nch):
    f = pl.kernel(
        _make_prop_body(nch),
        out_type=jax.ShapeDtypeStruct((NC, NP2, HID), jnp.float32),
        mesh=_MESH,
        compiler_params=_SC_PARAMS,
        scratch_types=[
            pltpu.VMEM_SHARED((NP2, HID), jnp.float32),
            pltpu.VMEM((NBUF, C), jnp.int32),
            pltpu.VMEM((NBUF, C), jnp.int32),
            pltpu.VMEM((NBUF, C), jnp.float32),
            pltpu.VMEM((C, HID), jnp.float32),
            pltpu.VMEM((C, HID), jnp.float32),
            pltpu.VMEM((C, HID), jnp.float32),
            pltpu.VMEM((C, HID), jnp.float32),
            pltpu.SemaphoreType.DMA,
            pltpu.SemaphoreType.DMA,
            pltpu.SemaphoreType.DMA,
            pltpu.SemaphoreType.DMA,
            pltpu.SemaphoreType.DMA,
            pltpu.SemaphoreType.DMA,
            pltpu.SemaphoreType.DMA,
            pltpu.SemaphoreType.DMA,
            pltpu.SemaphoreType.DMA,
            pltpu.SemaphoreType.DMA,
            pltpu.SemaphoreType.DMA,
            pltpu.SemaphoreType.DMA,
            pltpu.SemaphoreType.DMA,
            pltpu.SemaphoreType.DMA,
            pltpu.SemaphoreType.DMA,
            pltpu.SemaphoreType.DMA,
        ],
    )
    return f(table, srcF, dstF, nrmF)


def _dinv_body(part_ref, dinv_ref, selfnorm_ref):
    deg = jnp.sum(part_ref[...], axis=0, keepdims=True) + 1.0
    dinv = jax.lax.rsqrt(deg)
    dinv_ref[...] = dinv
    selfnorm_ref[...] = dinv * dinv


def _k_dinv(partials):
    return pl.pallas_call(
        _dinv_body,
        out_shape=[
            jax.ShapeDtypeStruct((1, N), jnp.float32),
            jax.ShapeDtypeStruct((1, N), jnp.float32),
        ],
    )(partials)


def _mm_body(x_ref, w_ref, o_ref):
    o_ref[...] = jax.lax.dot_general(
        x_ref[...], w_ref[...], (((1,), (0,)), ((), ())),
        preferred_element_type=jnp.float32)


def _k_mm(x, w):
    return pl.pallas_call(
        _mm_body,
        out_shape=jax.ShapeDtypeStruct((x.shape[0], w.shape[1]), jnp.float32),
    )(x, w)


def _bias_mm_body(o_ref, b_ref, w_ref, h_ref):
    h = o_ref[:, :N, :].reshape(NC * N, HID) + b_ref[...]
    h_ref[...] = jax.lax.dot_general(
        h, w_ref[...], (((1,), (0,)), ((), ())),
        preferred_element_type=jnp.float32)


def _k_bias_mm(o, b, w):
    return pl.pallas_call(
        _bias_mm_body,
        out_shape=jax.ShapeDtypeStruct((NC * N, HID), jnp.float32),
    )(o, b.reshape(1, HID), w)


def _bias_mm_sn_body(o_ref, t_ref, sn_ref, b_ref, w_ref, h_ref):
    h = (o_ref[:, :N, :].reshape(NC * N, HID)
         + jnp.tile(sn_ref[...], (NC, 1)) * t_ref[...]
         + b_ref[...])
    h_ref[...] = jax.lax.dot_general(
        h, w_ref[...], (((1,), (0,)), ((), ())),
        preferred_element_type=jnp.float32)


def _k_bias_mm_sn(o, t, sn_col, b, w):
    return pl.pallas_call(
        _bias_mm_sn_body,
        out_shape=jax.ShapeDtypeStruct((NC * N, HID), jnp.float32),
    )(o, t, sn_col, b.reshape(1, HID), w)


def _post_body(o_ref, t_ref, sn_ref, b_ref, a_ref, wc_ref,
               z_ref, hdec_ref, sum_ref):
    h = (o_ref[:, :N, :]
         + (jnp.tile(sn_ref[...], (NC, 1)) * t_ref[...]).reshape(NC, N, HID)
         + b_ref[...][None])
    z = jnp.where(h >= 0, h, a_ref[...][None] * h)
    z_ref[...] = z
    zp = z[0]
    hdec_ref[...] = jax.lax.dot_general(
        zp, wc_ref[...], (((1,), (0,)), ((), ())),
        preferred_element_type=jnp.float32)
    m = jnp.mean(zp, axis=0, keepdims=True)
    sum_ref[...] = 1.0 / (1.0 + jnp.exp(-m))


def _k_post(o, t, sn_col, b, a, wc):
    return pl.pallas_call(
        _post_body,
        out_shape=[
            jax.ShapeDtypeStruct((NC, N, HID), jnp.float32),
            jax.ShapeDtypeStruct((N, HID), jnp.float32),
            jax.ShapeDtypeStruct((1, HID), jnp.float32),
        ],
    )(o, t, sn_col, b.reshape(1, HID), a.reshape(1, HID), wc)


def _fin_body(p_ref, t_ref, sn_ref, bc_ref, mu_ref, xr_ref, q_ref):
    xr = (p_ref[0, :N, :] + p_ref[1, :N, :]
          + sn_ref[...] * t_ref[...] + bc_ref[...])
    xr_ref[...] = xr
    mu = mu_ref[...]
    d2 = (jnp.sum(xr * xr, axis=1, keepdims=True)
          + jnp.sum(mu * mu, axis=1)[None, :]
          - 2.0 * jax.lax.dot_general(xr, mu, (((1,), (1,)), ((), ())),
                                      preferred_element_type=jnp.float32))
    q = 1.0 / (1.0 + d2 / ALPHA + 1e-08)
    q = q ** (ALPHA + 1.0) / 2.0
    q_ref[...] = q / jnp.sum(q, axis=1, keepdims=True)


def _k_fin(p, t, sn_col, bc, mu):
    K = mu.shape[0]
    return pl.pallas_call(
        _fin_body,
        out_shape=[
            jax.ShapeDtypeStruct((N, HID), jnp.float32),
            jax.ShapeDtypeStruct((N, K), jnp.float32),
        ],
    )(p, t, sn_col, bc.reshape(1, HID), mu)


def kernel(x, edge_index, edge_attr, W1, b1, W2, b2, W3, b3, W4, b4, prelu_a, Wc, bc, mu):
    src = edge_index[0].astype(jnp.int32)
    dst = edge_index[1].astype(jnp.int32)
    perm = _dgi_perm()

    partials = _k_deg(dst, edge_attr)
    dinv2d, selfnorm2d = _k_dinv(partials)
    dinv = dinv2d[0]
    selfnorm = selfnorm2d[0]
    norm_e, psrc = _k_norm(src, dst, edge_attr, dinv, perm)

    sn_col = selfnorm.reshape(N, 1)

    sl = jnp.arange(N, dtype=jnp.int32)
    pz1 = jnp.zeros((EP - E2,), jnp.int32)
    pf1 = jnp.zeros((EP - E2,), jnp.float32)
    pzr = jnp.zeros((EPR - E,), jnp.int32)
    pfr = jnp.zeros((EPR - E,), jnp.float32)
    pzd = jnp.zeros((EPD - E,), jnp.int32)
    pfd = jnp.zeros((EPD - E,), jnp.float32)

    srcI1 = jnp.concatenate([src, sl, pz1, psrc, perm, pz1])
    dstI1 = jnp.concatenate([dst, sl, pz1, dst, sl, pz1])
    nrmI1 = jnp.concatenate([norm_e, selfnorm, pf1, norm_e, selfnorm, pf1])

    srcIR = jnp.concatenate([src, pzr, src + N, pzr])
    dstIR = jnp.concatenate([dst, pzr, dst, pzr])
    nrmIR = jnp.concatenate([norm_e, pfr, norm_e, pfr])

    srcD = jnp.concatenate([src, pzd])
    dstD = jnp.concatenate([dst, pzd])
    nrmD = jnp.concatenate([norm_e, pfd])

    xw = _k_mm(x, W1)
    o1 = _k_prop(xw, srcI1, dstI1, nrmI1, NCH)
    h2 = _k_bias_mm(o1, b1, W2)
    o2 = _k_prop(h2, srcIR, dstIR, nrmIR, NCHR)
    h3 = _k_bias_mm_sn(o2, h2, sn_col, b2, W3)
    o3 = _k_prop(h3, srcIR, dstIR, nrmIR, NCHR)
    h4 = _k_bias_mm_sn(o3, h3, sn_col, b3, W4)
    o4 = _k_prop(h4, srcIR, dstIR, nrmIR, NCHR)
    z2, hdec, sum2d = _k_post(o4, h4, sn_col, b4, prelu_a, Wc)
    od = _k_prop(hdec, srcD, dstD, nrmD, NCHD)
    xr, q = _k_fin(od, hdec, sn_col, bc, mu)

    pos_z = z2[0]
    neg_z = z2[1]
    summary = sum2d[0]
    return (pos_z, neg_z, summary, xr, q)
